# Initial kernel scaffold; baseline (speedup 1.0000x reference)
import functools

import jax
import jax.numpy as jnp
from jax import lax
from jax.experimental import pallas as pl
from jax.experimental.pallas import tpu as pltpu
from jax.experimental.pallas import tpu_sc as plsc

N = 10000
E = 320000
D = 128
NC = 2
NS = 16
NW = NC * NS
EPW = E // NW
C = 80
NCHUNK = EPW // C
RPT = N // NS
KA = 5000
KB = 2500

_MESH = plsc.VectorSubcoreMesh(core_axis_name="c", subcore_axis_name="s")
_Z16 = jnp.zeros((16,), jnp.float32)


@functools.partial(
    pl.kernel,
    out_type=jax.ShapeDtypeStruct((NW, 2, N), jnp.float32),
    mesh=_MESH,
    scratch_types=[
        pltpu.VMEM((EPW,), jnp.int32),
        pltpu.VMEM((EPW,), jnp.int32),
        pltpu.VMEM((N,), jnp.float32),
        pltpu.VMEM((2, N), jnp.float32),
    ],
)
def _sc_degrees(src_hbm, dst_hbm, keep_hbm, out_hbm, src_v, dst_v, keep_v, deg_v):
    wid = lax.axis_index("s") * NC + lax.axis_index("c")
    base = pl.multiple_of(wid * EPW, 8)
    pltpu.sync_copy(src_hbm.at[pl.ds(base, EPW)], src_v)
    pltpu.sync_copy(dst_hbm.at[pl.ds(base, EPW)], dst_v)
    pltpu.sync_copy(keep_hbm, keep_v)

    def zbody(i, _):
        deg_v[0, pl.ds(i * 16, 16)] = _Z16
        deg_v[1, pl.ds(i * 16, 16)] = _Z16
        return 0

    lax.fori_loop(0, N // 16, zbody, 0)

    def ebody(g, _):
        sl = pl.ds(g * 16, 16)
        s = src_v[sl]
        d = dst_v[sl]
        m = plsc.load_gather(keep_v, [s]) * plsc.load_gather(keep_v, [d])
        plsc.addupdate_scatter(deg_v.at[0], [s], m)
        plsc.addupdate_scatter(deg_v.at[1], [d], m)
        return 0

    lax.fori_loop(0, EPW // 16, ebody, 0)
    pltpu.sync_copy(deg_v, out_hbm.at[wid])


@functools.partial(
    pl.kernel,
    out_type=jax.ShapeDtypeStruct((NC, N, D), jnp.float32),
    mesh=_MESH,
    scratch_types=[
        pltpu.VMEM((EPW,), jnp.int32),
        pltpu.VMEM((EPW,), jnp.int32),
        pltpu.VMEM((EPW,), jnp.float32),
        pltpu.VMEM((N,), jnp.float32),
        pltpu.VMEM((N,), jnp.float32),
        pltpu.VMEM((EPW,), jnp.float32),
        pltpu.VMEM((C,), jnp.int32),
        pltpu.VMEM((C,), jnp.int32),
        pltpu.VMEM((C, D), jnp.float32),
        pltpu.VMEM_SHARED((N, D), jnp.float32),
        pltpu.SemaphoreType.DMA,
    ],
)
def _sc_edge_pass(h2_hbm, src_hbm, dst_hbm, ew_hbm, norm_hbm, keep_hbm, out_hbm,
                  srcs_v, dsts_v, ews_v, norm_v, keep_v, coef_v, srcc_v, dstc_v,
                  rows_v, agg_sh, sem):
    cid = lax.axis_index("c")
    sid = lax.axis_index("s")
    wid = sid * NC + cid
    base = pl.multiple_of(wid * EPW, 8)
    pltpu.sync_copy(src_hbm.at[pl.ds(base, EPW)], srcs_v)
    pltpu.sync_copy(dst_hbm.at[pl.ds(base, EPW)], dsts_v)
    pltpu.sync_copy(ew_hbm.at[pl.ds(base, EPW)], ews_v)
    pltpu.sync_copy(norm_hbm, norm_v)
    pltpu.sync_copy(keep_hbm, keep_v)

    def cbody(g, _):
        sl = pl.ds(g * 16, 16)
        s = srcs_v[sl]
        d = dsts_v[sl]
        no = plsc.load_gather(norm_v, [s])
        ks = plsc.load_gather(keep_v, [s])
        kd = plsc.load_gather(keep_v, [d])
        coef_v[sl] = no * ews_v[sl] * ks * kd
        return 0

    lax.fori_loop(0, EPW // 16, cbody, 0)

    def z0(i, _):
        rows_v[i // 8, pl.ds((i % 8) * 16, 16)] = _Z16
        return 0

    lax.fori_loop(0, 25 * 8, z0, 0)

    def z1(i, _):
        pltpu.sync_copy(rows_v.at[pl.ds(0, 25)],
                        agg_sh.at[pl.ds(sid * RPT + i * 25, 25)])
        return 0

    lax.fori_loop(0, RPT // 25, z1, 0)
    plsc.subcore_barrier()

    def chunk(i, _):
        off = pl.multiple_of(base + i * C, 8)
        pltpu.sync_copy(src_hbm.at[pl.ds(off, C)], srcc_v)
        pltpu.sync_copy(dst_hbm.at[pl.ds(off, C)], dstc_v)
        pltpu.async_copy(h2_hbm.at[srcc_v], rows_v, sem).wait()

        def sbody(g, _):
            c = coef_v[pl.ds(i * C + g * 16, 16)]
            eids = lax.iota(jnp.int32, 16) + g * 16

            def dbody(dd, _):
                dvec = jnp.full((16,), dd, jnp.int32)
                v = plsc.load_gather(rows_v, [eids, dvec])
                plsc.store_scatter(rows_v, [eids, dvec], v * c)
                return 0

            lax.fori_loop(0, D, dbody, 0)
            return 0

        lax.fori_loop(0, C // 16, sbody, 0)
        pltpu.sync_copy(rows_v, agg_sh.at[dstc_v], add=True)
        return 0

    lax.fori_loop(0, NCHUNK, chunk, 0)
    plsc.subcore_barrier()
    r0 = sid * RPT
    pltpu.sync_copy(agg_sh.at[pl.ds(r0, RPT)], out_hbm.at[cid, pl.ds(r0, RPT)])


@functools.partial(
    pl.kernel,
    out_type=jax.ShapeDtypeStruct((NW, N), jnp.float32),
    mesh=_MESH,
    scratch_types=[
        pltpu.VMEM((EPW,), jnp.int32),
        pltpu.VMEM((EPW,), jnp.int32),
        pltpu.VMEM((EPW,), jnp.float32),
        pltpu.VMEM((N,), jnp.float32),
        pltpu.VMEM((N,), jnp.float32),
        pltpu.VMEM((N,), jnp.float32),
        pltpu.VMEM((N,), jnp.float32),
    ],
)
def _sc_score_pass(src_hbm, dst_hbm, ew_hbm, s2_hbm, norm_hbm, keep_hbm, out_hbm,
                   src_v, dst_v, ew_v, s2_v, norm_v, keep_v, sagg_v):
    wid = lax.axis_index("s") * NC + lax.axis_index("c")
    base = pl.multiple_of(wid * EPW, 8)
    pltpu.sync_copy(src_hbm.at[pl.ds(base, EPW)], src_v)
    pltpu.sync_copy(dst_hbm.at[pl.ds(base, EPW)], dst_v)
    pltpu.sync_copy(ew_hbm.at[pl.ds(base, EPW)], ew_v)
    pltpu.sync_copy(s2_hbm, s2_v)
    pltpu.sync_copy(norm_hbm, norm_v)
    pltpu.sync_copy(keep_hbm, keep_v)

    def zbody(i, _):
        sagg_v[pl.ds(i * 16, 16)] = _Z16
        return 0

    lax.fori_loop(0, N // 16, zbody, 0)

    def ebody(g, _):
        sl = pl.ds(g * 16, 16)
        s = src_v[sl]
        d = dst_v[sl]
        val = (plsc.load_gather(s2_v, [s]) * plsc.load_gather(norm_v, [s])
               * ew_v[sl] * plsc.load_gather(keep_v, [s])
               * plsc.load_gather(keep_v, [d]))
        plsc.addupdate_scatter(sagg_v, [d], val)
        return 0

    lax.fori_loop(0, EPW // 16, ebody, 0)
    pltpu.sync_copy(sagg_v, out_hbm.at[wid])


def _sortable(x):
    b = lax.bitcast_convert_type(x, jnp.uint32)
    return jnp.where((b >> jnp.uint32(31)) != jnp.uint32(0),
                     ~b, b | jnp.uint32(0x80000000))


def _kth_thr(m, k):
    thr = jnp.uint32(0)
    for i in range(31, -1, -1):
        cand = thr | jnp.uint32(1 << i)
        cnt = jnp.sum((m >= cand).astype(jnp.int32))
        thr = jnp.where(cnt >= k, cand, thr)
    return thr


def _tc_norms_h2_body(degp, feat, W, norms, h2):
    deg = jnp.sum(degp[...], axis=0)
    norms[...] = lax.rsqrt(jnp.clip(deg, 1.0, None))
    h2[...] = jnp.dot(feat[...], W[...], preferred_element_type=jnp.float32)


_tc_norms_h2 = pl.pallas_call(
    _tc_norms_h2_body,
    out_shape=(jax.ShapeDtypeStruct((2, N), jnp.float32),
               jax.ShapeDtypeStruct((N, D), jnp.float32)),
)


def _tc_norms_body(degp, norms):
    deg = jnp.sum(degp[...], axis=0)
    norms[...] = lax.rsqrt(jnp.clip(deg, 1.0, None))


_tc_norms = pl.pallas_call(
    _tc_norms_body, out_shape=jax.ShapeDtypeStruct((2, N), jnp.float32))


def _tc_out_s2_body(aggp, norms, b, Wp, out, s2):
    agg = aggp[0] + aggp[1]
    o = jnp.maximum(agg * norms[1][:, None] + b[...][None, :], 0.0)
    out[...] = o
    s2[...] = jnp.dot(o, Wp[...], preferred_element_type=jnp.float32)


_tc_out_s2 = pl.pallas_call(
    _tc_out_s2_body,
    out_shape=(jax.ShapeDtypeStruct((N, D), jnp.float32),
               jax.ShapeDtypeStruct((N, 1), jnp.float32)),
)


def _tc_pool0_body(saggp, norms, bp, out0, W1, keep_o, h2o, r0o):
    score = jnp.sum(saggp[...], axis=0) * norms[1] + bp[0]
    m = _sortable(score)
    thr = _kth_thr(m, KA)
    keep = m >= thr
    keepf = keep.astype(jnp.float32)
    nf = out0[...] * (jnp.tanh(score) * keepf)[:, None]
    keep_o[...] = keepf
    h2o[...] = jnp.dot(nf, W1[...], preferred_element_type=jnp.float32)
    rmean = jnp.sum(nf, axis=0) * (1.0 / KA)
    rmax = jnp.max(jnp.where(keep[:, None], nf, -jnp.inf), axis=0)
    r0o[...] = jnp.concatenate([rmean, rmax])


_tc_pool0 = pl.pallas_call(
    _tc_pool0_body,
    out_shape=(jax.ShapeDtypeStruct((N,), jnp.float32),
               jax.ShapeDtypeStruct((N, D), jnp.float32),
               jax.ShapeDtypeStruct((2 * D,), jnp.float32)),
)


def _tc_final_body(saggp, norms, bp, out1, keep0, r0,
                   W1, b1, W2, b2, W3, b3, out):
    score = jnp.sum(saggp[...], axis=0) * norms[1] + bp[0]
    m = _sortable(score)
    m = jnp.where(keep0[...] > 0.0, m, jnp.uint32(0))
    thr = _kth_thr(m, KB)
    keep = m >= thr
    keepf = keep.astype(jnp.float32)
    nf = out1[...] * (jnp.tanh(score) * keepf)[:, None]
    rmean = jnp.sum(nf, axis=0) * (1.0 / KB)
    rmax = jnp.max(jnp.where(keep[:, None], nf, -jnp.inf), axis=0)
    fr = (r0[...] + jnp.concatenate([rmean, rmax])).reshape(1, 2 * D)
    h = jnp.maximum(jnp.dot(fr, W1[...]) + b1[...][None, :], 0.0)
    h = jnp.maximum(jnp.dot(h, W2[...]) + b2[...][None, :], 0.0)
    z = jnp.dot(h, W3[...]) + b3[...][None, :]
    zm = z - jnp.max(z, axis=-1, keepdims=True)
    out[...] = zm - jnp.log(jnp.sum(jnp.exp(zm), axis=-1, keepdims=True))


_tc_final = pl.pallas_call(
    _tc_final_body, out_shape=jax.ShapeDtypeStruct((1, 2), jnp.float32))


def kernel(feat, edge_index, eweight, conv0_W, conv0_b, pool0_W, pool0_b,
           conv1_W, conv1_b, pool1_W, pool1_b, lin1_W, lin1_b, lin2_W, lin2_b,
           lin3_W, lin3_b):
    src = edge_index[0]
    dst = edge_index[1]
    ones = jnp.ones((N,), jnp.float32)

    degp0 = _sc_degrees(src, dst, ones)
    norms0, h2_0 = _tc_norms_h2(degp0, feat, conv0_W)
    no0 = norms0[0]
    aggp0 = _sc_edge_pass(h2_0, src, dst, eweight, no0, ones)
    out0, s2_0 = _tc_out_s2(aggp0, norms0, conv0_b, pool0_W)
    saggp0 = _sc_score_pass(src, dst, eweight, s2_0.reshape(N), no0, ones)
    keep0, h2_1, r0 = _tc_pool0(saggp0, norms0, pool0_b, out0, conv1_W)

    degp1 = _sc_degrees(src, dst, keep0)
    norms1 = _tc_norms(degp1)
    no1 = norms1[0]
    aggp1 = _sc_edge_pass(h2_1, src, dst, eweight, no1, keep0)
    out1, s2_1 = _tc_out_s2(aggp1, norms1, conv1_b, pool1_W)
    saggp1 = _sc_score_pass(src, dst, eweight, s2_1.reshape(N), no1, keep0)
    return _tc_final(saggp1, norms1, pool1_b, out1, keep0, r0,
                     lin1_W, lin1_b, lin2_W, lin2_b, lin3_W, lin3_b)


# speedup vs baseline: 5.6304x; 5.6304x over previous
import functools

import jax
import jax.numpy as jnp
from jax import lax
from jax.experimental import pallas as pl
from jax.experimental.pallas import tpu as pltpu
from jax.experimental.pallas import tpu_sc as plsc

N = 10000
E = 320000
D = 128
NC = 2
NS = 16
NW = NC * NS
EPW = E // NW
C = 80
NCHUNK = EPW // C
TR = 624
KA = 5000
KB = 2500

_MESH = plsc.VectorSubcoreMesh(core_axis_name="c", subcore_axis_name="s")


def _z16():
    return jnp.zeros((16,), jnp.float32)


@functools.partial(
    pl.kernel,
    out_type=(jax.ShapeDtypeStruct((NW * N,), jnp.float32),
              jax.ShapeDtypeStruct((NW * N,), jnp.float32)),
    mesh=_MESH,
    compiler_params=pltpu.CompilerParams(needs_layout_passes=False),
    scratch_types=[
        pltpu.VMEM((EPW,), jnp.int32),
        pltpu.VMEM((EPW,), jnp.int32),
        pltpu.VMEM((N,), jnp.float32),
        pltpu.VMEM((N,), jnp.float32),
        pltpu.VMEM((N,), jnp.float32),
    ],
)
def _sc_degrees(src_hbm, dst_hbm, keep_hbm, dego_hbm, degi_hbm,
                src_v, dst_v, keep_v, dego_v, degi_v):
    wid = lax.axis_index("s") * NC + lax.axis_index("c")
    base = pl.multiple_of(wid * EPW, 8)
    pltpu.sync_copy(src_hbm.at[pl.ds(base, EPW)], src_v)
    pltpu.sync_copy(dst_hbm.at[pl.ds(base, EPW)], dst_v)
    pltpu.sync_copy(keep_hbm, keep_v)

    def zbody(i, _):
        dego_v[pl.ds(i * 16, 16)] = _z16()
        degi_v[pl.ds(i * 16, 16)] = _z16()
        return 0

    lax.fori_loop(0, N // 16, zbody, 0)

    def ebody(g, _):
        sl = pl.ds(g * 16, 16)
        s = src_v[sl]
        d = dst_v[sl]
        m = plsc.load_gather(keep_v, [s]) * plsc.load_gather(keep_v, [d])
        plsc.addupdate_scatter(dego_v, [s], m)
        plsc.addupdate_scatter(degi_v, [d], m)
        return 0

    lax.fori_loop(0, EPW // 16, ebody, 0)
    obase = pl.multiple_of(wid * N, 8)
    pltpu.sync_copy(dego_v, dego_hbm.at[pl.ds(obase, N)])
    pltpu.sync_copy(degi_v, degi_hbm.at[pl.ds(obase, N)])


@functools.partial(
    pl.kernel,
    out_type=jax.ShapeDtypeStruct((NC, N, D), jnp.float32),
    mesh=_MESH,
    compiler_params=pltpu.CompilerParams(needs_layout_passes=False),
    scratch_types=[
        pltpu.VMEM((N,), jnp.float32),
        pltpu.VMEM((N,), jnp.float32),
        pltpu.VMEM((C,), jnp.int32),
        pltpu.VMEM((C,), jnp.int32),
        pltpu.VMEM((C,), jnp.float32),
        pltpu.VMEM((C,), jnp.float32),
        pltpu.VMEM((C, D), jnp.float32),
        pltpu.VMEM_SHARED((N, D), jnp.float32),
        pltpu.SemaphoreType.DMA,
    ],
)
def _sc_edge_pass(h2_hbm, src_hbm, dst_hbm, ew_hbm, norm_hbm, keep_hbm, out_hbm,
                  norm_v, keep_v, srcc_v, dstc_v, ewc_v, coef_v, rows_v,
                  agg_sh, sem):
    cid = lax.axis_index("c")
    sid = lax.axis_index("s")
    wid = sid * NC + cid
    base = pl.multiple_of(wid * EPW, 8)
    pltpu.sync_copy(norm_hbm, norm_v)
    pltpu.sync_copy(keep_hbm, keep_v)

    def z0(i, _):
        rows_v[i // 8, pl.ds((i % 8) * 16, 16)] = _z16()
        return 0

    lax.fori_loop(0, 16 * 8, z0, 0)
    zslab = rows_v.at[pl.ds(0, 16)]

    def z1(i, _):
        pltpu.sync_copy(zslab, agg_sh.at[pl.ds(sid * TR + i * 16, 16)])
        return 0

    lax.fori_loop(0, TR // 16, z1, 0)

    @pl.when(sid == NS - 1)
    def _():
        pltpu.sync_copy(zslab, agg_sh.at[pl.ds(NS * TR, N - NS * TR)])

    plsc.subcore_barrier()

    def chunk(i, _):
        off = pl.multiple_of(base + i * C, 8)
        pltpu.sync_copy(src_hbm.at[pl.ds(off, C)], srcc_v)
        pltpu.sync_copy(dst_hbm.at[pl.ds(off, C)], dstc_v)
        pltpu.sync_copy(ew_hbm.at[pl.ds(off, C)], ewc_v)
        pltpu.async_copy(h2_hbm.at[srcc_v], rows_v, sem).wait()

        def cbody(g, _):
            sl = pl.ds(g * 16, 16)
            s = srcc_v[sl]
            d = dstc_v[sl]
            no = plsc.load_gather(norm_v, [s])
            ks = plsc.load_gather(keep_v, [s])
            kd = plsc.load_gather(keep_v, [d])
            coef_v[sl] = no * ewc_v[sl] * ks * kd
            return 0

        lax.fori_loop(0, C // 16, cbody, 0)

        def sbody(g, _):
            c = coef_v[pl.ds(g * 16, 16)]
            eids = lax.iota(jnp.int32, 16) + g * 16

            def dbody(dd, _):
                dvec = jnp.full((16,), dd, jnp.int32)
                v = plsc.load_gather(rows_v, [eids, dvec])
                plsc.store_scatter(rows_v, [eids, dvec], v * c)
                return 0

            lax.fori_loop(0, D, dbody, 0)
            return 0

        lax.fori_loop(0, C // 16, sbody, 0)
        pltpu.sync_copy(rows_v, agg_sh.at[dstc_v], add=True)
        return 0

    lax.fori_loop(0, NCHUNK, chunk, 0)
    plsc.subcore_barrier()
    r0 = sid * TR
    pltpu.sync_copy(agg_sh.at[pl.ds(r0, TR)], out_hbm.at[cid, pl.ds(r0, TR)])

    @pl.when(sid == NS - 1)
    def _():
        pltpu.sync_copy(agg_sh.at[pl.ds(NS * TR, N - NS * TR)],
                        out_hbm.at[cid, pl.ds(NS * TR, N - NS * TR)])


@functools.partial(
    pl.kernel,
    out_type=jax.ShapeDtypeStruct((NW * N,), jnp.float32),
    mesh=_MESH,
    compiler_params=pltpu.CompilerParams(needs_layout_passes=False),
    scratch_types=[
        pltpu.VMEM((EPW,), jnp.int32),
        pltpu.VMEM((EPW,), jnp.int32),
        pltpu.VMEM((EPW,), jnp.float32),
        pltpu.VMEM((N,), jnp.float32),
        pltpu.VMEM((N,), jnp.float32),
        pltpu.VMEM((N,), jnp.float32),
        pltpu.VMEM((N,), jnp.float32),
    ],
)
def _sc_score_pass(src_hbm, dst_hbm, ew_hbm, s2_hbm, norm_hbm, keep_hbm, out_hbm,
                   src_v, dst_v, ew_v, s2_v, norm_v, keep_v, sagg_v):
    wid = lax.axis_index("s") * NC + lax.axis_index("c")
    base = pl.multiple_of(wid * EPW, 8)
    pltpu.sync_copy(src_hbm.at[pl.ds(base, EPW)], src_v)
    pltpu.sync_copy(dst_hbm.at[pl.ds(base, EPW)], dst_v)
    pltpu.sync_copy(ew_hbm.at[pl.ds(base, EPW)], ew_v)
    pltpu.sync_copy(s2_hbm, s2_v)
    pltpu.sync_copy(norm_hbm, norm_v)
    pltpu.sync_copy(keep_hbm, keep_v)

    def zbody(i, _):
        sagg_v[pl.ds(i * 16, 16)] = _z16()
        return 0

    lax.fori_loop(0, N // 16, zbody, 0)

    def ebody(g, _):
        sl = pl.ds(g * 16, 16)
        s = src_v[sl]
        d = dst_v[sl]
        val = (plsc.load_gather(s2_v, [s]) * plsc.load_gather(norm_v, [s])
               * ew_v[sl] * plsc.load_gather(keep_v, [s])
               * plsc.load_gather(keep_v, [d]))
        plsc.addupdate_scatter(sagg_v, [d], val)
        return 0

    lax.fori_loop(0, EPW // 16, ebody, 0)
    obase = pl.multiple_of(wid * N, 8)
    pltpu.sync_copy(sagg_v, out_hbm.at[pl.ds(obase, N)])


def _sortable(x):
    b = lax.bitcast_convert_type(x, jnp.uint32)
    return jnp.where((b >> jnp.uint32(31)) != jnp.uint32(0),
                     ~b, b | jnp.uint32(0x80000000))


def _kth_thr(m, k):
    thr = jnp.uint32(0)
    for i in range(31, -1, -1):
        cand = thr | jnp.uint32(1 << i)
        cnt = jnp.sum((m >= cand).astype(jnp.int32))
        thr = jnp.where(cnt >= k, cand, thr)
    return thr


def _colT(x):
    return lax.transpose(x, (1, 0))


def _tc_norms_h2_body(dego, degi, feat, W, noc, nic, h2):
    noc[...] = _colT(lax.rsqrt(jnp.clip(
        jnp.sum(dego[...], axis=0, keepdims=True), 1.0, None)))
    nic[...] = _colT(lax.rsqrt(jnp.clip(
        jnp.sum(degi[...], axis=0, keepdims=True), 1.0, None)))
    h2[...] = jnp.dot(feat[...], W[...], preferred_element_type=jnp.float32)


_tc_norms_h2 = pl.pallas_call(
    _tc_norms_h2_body,
    out_shape=(jax.ShapeDtypeStruct((N, 1), jnp.float32),
               jax.ShapeDtypeStruct((N, 1), jnp.float32),
               jax.ShapeDtypeStruct((N, D), jnp.float32)),
)


def _tc_norms_body(dego, degi, noc, nic):
    noc[...] = _colT(lax.rsqrt(jnp.clip(
        jnp.sum(dego[...], axis=0, keepdims=True), 1.0, None)))
    nic[...] = _colT(lax.rsqrt(jnp.clip(
        jnp.sum(degi[...], axis=0, keepdims=True), 1.0, None)))


_tc_norms = pl.pallas_call(
    _tc_norms_body,
    out_shape=(jax.ShapeDtypeStruct((N, 1), jnp.float32),
               jax.ShapeDtypeStruct((N, 1), jnp.float32)),
)


def _tc_out_s2_body(aggp, nic, b, Wp, out, s2):
    o = jnp.maximum((aggp[0] + aggp[1]) * nic[...] + b[...][None, :], 0.0)
    out[...] = o
    s2[...] = jnp.dot(o, Wp[...], preferred_element_type=jnp.float32)


_tc_out_s2 = pl.pallas_call(
    _tc_out_s2_body,
    out_shape=(jax.ShapeDtypeStruct((N, D), jnp.float32),
               jax.ShapeDtypeStruct((N, 1), jnp.float32)),
)


def _tc_pool0_body(saggp, nic, bp, out0, W1, keep_o, h2o, r0o):
    score = _colT(jnp.sum(saggp[...], axis=0, keepdims=True)) * nic[...] + bp[0]
    m = _sortable(score)
    thr = _kth_thr(m, KA)
    keep = m >= thr
    keepf = keep.astype(jnp.float32)
    nf = out0[...] * (jnp.tanh(score) * keepf)
    keep_o[...] = keepf
    h2o[...] = jnp.dot(nf, W1[...], preferred_element_type=jnp.float32)
    rmean = jnp.sum(nf, axis=0, keepdims=True) * (1.0 / KA)
    rmax = jnp.max(jnp.where(keep, nf, -jnp.inf), axis=0, keepdims=True)
    r0o[...] = jnp.concatenate([rmean, rmax], axis=1)


_tc_pool0 = pl.pallas_call(
    _tc_pool0_body,
    out_shape=(jax.ShapeDtypeStruct((N, 1), jnp.float32),
               jax.ShapeDtypeStruct((N, D), jnp.float32),
               jax.ShapeDtypeStruct((1, 2 * D), jnp.float32)),
)


def _tc_final_body(saggp, nic, bp, out1, keep0, r0,
                   W1, b1, W2, b2, W3, b3, out):
    score = _colT(jnp.sum(saggp[...], axis=0, keepdims=True)) * nic[...] + bp[0]
    m = _sortable(score)
    m = jnp.where(keep0[...] > 0.0, m, jnp.uint32(0))
    thr = _kth_thr(m, KB)
    keep = m >= thr
    keepf = keep.astype(jnp.float32)
    nf = out1[...] * (jnp.tanh(score) * keepf)
    rmean = jnp.sum(nf, axis=0, keepdims=True) * (1.0 / KB)
    rmax = jnp.max(jnp.where(keep, nf, -jnp.inf), axis=0, keepdims=True)
    fr = r0[...] + jnp.concatenate([rmean, rmax], axis=1)
    h = jnp.maximum(jnp.dot(fr, W1[...]) + b1[...][None, :], 0.0)
    h = jnp.maximum(jnp.dot(h, W2[...]) + b2[...][None, :], 0.0)
    z = jnp.dot(h, W3[...]) + b3[...][None, :]
    zm = z - jnp.max(z, axis=-1, keepdims=True)
    out[...] = zm - jnp.log(jnp.sum(jnp.exp(zm), axis=-1, keepdims=True))


_tc_final = pl.pallas_call(
    _tc_final_body, out_shape=jax.ShapeDtypeStruct((1, 2), jnp.float32))


def kernel(feat, edge_index, eweight, conv0_W, conv0_b, pool0_W, pool0_b,
           conv1_W, conv1_b, pool1_W, pool1_b, lin1_W, lin1_b, lin2_W, lin2_b,
           lin3_W, lin3_b):
    src = edge_index[0]
    dst = edge_index[1]
    ones = jnp.ones((N,), jnp.float32)

    dego0, degi0 = _sc_degrees(src, dst, ones)
    noc0, nic0, h2_0 = _tc_norms_h2(dego0.reshape(NW, N), degi0.reshape(NW, N),
                                    feat, conv0_W)
    no0 = noc0.reshape(N)
    aggp0 = _sc_edge_pass(h2_0, src, dst, eweight, no0, ones)
    out0, s2_0 = _tc_out_s2(aggp0, nic0, conv0_b, pool0_W)
    saggp0 = _sc_score_pass(src, dst, eweight, s2_0.reshape(N), no0, ones)
    keep0, h2_1, r0 = _tc_pool0(saggp0.reshape(NW, N), nic0, pool0_b,
                                out0, conv1_W)
    keep0f = keep0.reshape(N)

    dego1, degi1 = _sc_degrees(src, dst, keep0f)
    noc1, nic1 = _tc_norms(dego1.reshape(NW, N), degi1.reshape(NW, N))
    no1 = noc1.reshape(N)
    aggp1 = _sc_edge_pass(h2_1, src, dst, eweight, no1, keep0f)
    out1, s2_1 = _tc_out_s2(aggp1, nic1, conv1_b, pool1_W)
    saggp1 = _sc_score_pass(src, dst, eweight, s2_1.reshape(N), no1, keep0f)
    return _tc_final(saggp1.reshape(NW, N), nic1, pool1_b, out1, keep0, r0,
                     lin1_W, lin1_b, lin2_W, lin2_b, lin3_W, lin3_b)


# speedup vs baseline: 18.7164x; 3.3242x over previous
import functools

import jax
import jax.numpy as jnp
from jax import lax
from jax.experimental import pallas as pl
from jax.experimental.pallas import tpu as pltpu
from jax.experimental.pallas import tpu_sc as plsc

N = 10000
E = 320000
D = 128
NC = 2
NS = 16
NW = NC * NS
EPW = E // NW
C = 80
NCHUNK = EPW // C
TR = 624
KA = 5000
KB = 2500

_MESH = plsc.VectorSubcoreMesh(core_axis_name="c", subcore_axis_name="s")


def _z16():
    return jnp.zeros((16,), jnp.float32)


@functools.partial(
    pl.kernel,
    out_type=(jax.ShapeDtypeStruct((NW * N,), jnp.float32),
              jax.ShapeDtypeStruct((NW * N,), jnp.float32)),
    mesh=_MESH,
    compiler_params=pltpu.CompilerParams(needs_layout_passes=False),
    scratch_types=[
        pltpu.VMEM((EPW,), jnp.int32),
        pltpu.VMEM((EPW,), jnp.int32),
        pltpu.VMEM((N,), jnp.float32),
        pltpu.VMEM((N,), jnp.float32),
        pltpu.VMEM((N,), jnp.float32),
    ],
)
def _sc_degrees(src_hbm, dst_hbm, keep_hbm, dego_hbm, degi_hbm,
                src_v, dst_v, keep_v, dego_v, degi_v):
    wid = lax.axis_index("s") * NC + lax.axis_index("c")
    base = pl.multiple_of(wid * EPW, 8)
    pltpu.sync_copy(src_hbm.at[pl.ds(base, EPW)], src_v)
    pltpu.sync_copy(dst_hbm.at[pl.ds(base, EPW)], dst_v)
    pltpu.sync_copy(keep_hbm, keep_v)

    def zbody(i, _):
        dego_v[pl.ds(i * 16, 16)] = _z16()
        degi_v[pl.ds(i * 16, 16)] = _z16()
        return 0

    lax.fori_loop(0, N // 16, zbody, 0)

    def ebody(g, _):
        sl = pl.ds(g * 16, 16)
        s = src_v[sl]
        d = dst_v[sl]
        m = plsc.load_gather(keep_v, [s]) * plsc.load_gather(keep_v, [d])
        plsc.addupdate_scatter(dego_v, [s], m)
        plsc.addupdate_scatter(degi_v, [d], m)
        return 0

    lax.fori_loop(0, EPW // 16, ebody, 0)
    obase = pl.multiple_of(wid * N, 8)
    pltpu.sync_copy(dego_v, dego_hbm.at[pl.ds(obase, N)])
    pltpu.sync_copy(degi_v, degi_hbm.at[pl.ds(obase, N)])


@functools.partial(
    pl.kernel,
    out_type=jax.ShapeDtypeStruct((NC, N, D), jnp.float32),
    mesh=_MESH,
    compiler_params=pltpu.CompilerParams(needs_layout_passes=False),
    scratch_types=[
        pltpu.VMEM((N,), jnp.float32),
        pltpu.VMEM((N,), jnp.float32),
        pltpu.VMEM((C,), jnp.int32),
        pltpu.VMEM((C,), jnp.int32),
        pltpu.VMEM((C,), jnp.float32),
        pltpu.VMEM((C,), jnp.float32),
        pltpu.VMEM((C, D), jnp.float32),
        pltpu.VMEM_SHARED((N, D), jnp.float32),
        pltpu.SemaphoreType.DMA,
    ],
)
def _sc_edge_pass(h2_hbm, src_hbm, dst_hbm, ew_hbm, norm_hbm, keep_hbm, out_hbm,
                  norm_v, keep_v, srcc_v, dstc_v, ewc_v, coef_v, rows_v,
                  agg_sh, sem):
    cid = lax.axis_index("c")
    sid = lax.axis_index("s")
    wid = sid * NC + cid
    base = pl.multiple_of(wid * EPW, 8)
    pltpu.sync_copy(norm_hbm, norm_v)
    pltpu.sync_copy(keep_hbm, keep_v)

    def z0(i, _):
        rows_v[i // 8, pl.ds((i % 8) * 16, 16)] = _z16()
        return 0

    lax.fori_loop(0, 16 * 8, z0, 0)
    zslab = rows_v.at[pl.ds(0, 16)]

    def z1(i, _):
        pltpu.sync_copy(zslab, agg_sh.at[pl.ds(sid * TR + i * 16, 16)])
        return 0

    lax.fori_loop(0, TR // 16, z1, 0)

    @pl.when(sid == NS - 1)
    def _():
        pltpu.sync_copy(zslab, agg_sh.at[pl.ds(NS * TR, N - NS * TR)])

    plsc.subcore_barrier()

    def chunk(i, _):
        off = pl.multiple_of(base + i * C, 8)
        pltpu.sync_copy(src_hbm.at[pl.ds(off, C)], srcc_v)
        pltpu.sync_copy(dst_hbm.at[pl.ds(off, C)], dstc_v)
        pltpu.sync_copy(ew_hbm.at[pl.ds(off, C)], ewc_v)
        pltpu.async_copy(h2_hbm.at[srcc_v], rows_v, sem).wait()

        def cbody(g, _):
            sl = pl.ds(g * 16, 16)
            s = srcc_v[sl]
            d = dstc_v[sl]
            no = plsc.load_gather(norm_v, [s])
            ks = plsc.load_gather(keep_v, [s])
            kd = plsc.load_gather(keep_v, [d])
            coef_v[sl] = no * ewc_v[sl] * ks * kd
            return 0

        lax.fori_loop(0, C // 16, cbody, 0)

        def sbody(e, _):
            cvec = plsc.load_gather(coef_v, [jnp.full((16,), e, jnp.int32)])
            for j in range(D // 16):
                sl = pl.ds(j * 16, 16)
                rows_v[e, sl] = rows_v[e, sl] * cvec
            return 0

        lax.fori_loop(0, C, sbody, 0)
        pltpu.sync_copy(rows_v, agg_sh.at[dstc_v], add=True)
        return 0

    lax.fori_loop(0, NCHUNK, chunk, 0)
    plsc.subcore_barrier()
    r0 = sid * TR
    pltpu.sync_copy(agg_sh.at[pl.ds(r0, TR)], out_hbm.at[cid, pl.ds(r0, TR)])

    @pl.when(sid == NS - 1)
    def _():
        pltpu.sync_copy(agg_sh.at[pl.ds(NS * TR, N - NS * TR)],
                        out_hbm.at[cid, pl.ds(NS * TR, N - NS * TR)])


@functools.partial(
    pl.kernel,
    out_type=jax.ShapeDtypeStruct((NW * N,), jnp.float32),
    mesh=_MESH,
    compiler_params=pltpu.CompilerParams(needs_layout_passes=False),
    scratch_types=[
        pltpu.VMEM((EPW,), jnp.int32),
        pltpu.VMEM((EPW,), jnp.int32),
        pltpu.VMEM((EPW,), jnp.float32),
        pltpu.VMEM((N,), jnp.float32),
        pltpu.VMEM((N,), jnp.float32),
        pltpu.VMEM((N,), jnp.float32),
        pltpu.VMEM((N,), jnp.float32),
    ],
)
def _sc_score_pass(src_hbm, dst_hbm, ew_hbm, s2_hbm, norm_hbm, keep_hbm, out_hbm,
                   src_v, dst_v, ew_v, s2_v, norm_v, keep_v, sagg_v):
    wid = lax.axis_index("s") * NC + lax.axis_index("c")
    base = pl.multiple_of(wid * EPW, 8)
    pltpu.sync_copy(src_hbm.at[pl.ds(base, EPW)], src_v)
    pltpu.sync_copy(dst_hbm.at[pl.ds(base, EPW)], dst_v)
    pltpu.sync_copy(ew_hbm.at[pl.ds(base, EPW)], ew_v)
    pltpu.sync_copy(s2_hbm, s2_v)
    pltpu.sync_copy(norm_hbm, norm_v)
    pltpu.sync_copy(keep_hbm, keep_v)

    def zbody(i, _):
        sagg_v[pl.ds(i * 16, 16)] = _z16()
        return 0

    lax.fori_loop(0, N // 16, zbody, 0)

    def ebody(g, _):
        sl = pl.ds(g * 16, 16)
        s = src_v[sl]
        d = dst_v[sl]
        val = (plsc.load_gather(s2_v, [s]) * plsc.load_gather(norm_v, [s])
               * ew_v[sl] * plsc.load_gather(keep_v, [s])
               * plsc.load_gather(keep_v, [d]))
        plsc.addupdate_scatter(sagg_v, [d], val)
        return 0

    lax.fori_loop(0, EPW // 16, ebody, 0)
    obase = pl.multiple_of(wid * N, 8)
    pltpu.sync_copy(sagg_v, out_hbm.at[pl.ds(obase, N)])


def _sortable(x):
    b = lax.bitcast_convert_type(x, jnp.uint32)
    return jnp.where((b >> jnp.uint32(31)) != jnp.uint32(0),
                     ~b, b | jnp.uint32(0x80000000))


def _kth_thr(m, k):
    thr = jnp.uint32(0)
    for i in range(31, -1, -1):
        cand = thr | jnp.uint32(1 << i)
        cnt = jnp.sum((m >= cand).astype(jnp.int32))
        thr = jnp.where(cnt >= k, cand, thr)
    return thr


def _colT(x):
    return lax.transpose(x, (1, 0))


def _tc_norms_h2_body(dego, degi, feat, W, noc, nic, h2):
    noc[...] = _colT(lax.rsqrt(jnp.clip(
        jnp.sum(dego[...], axis=0, keepdims=True), 1.0, None)))
    nic[...] = _colT(lax.rsqrt(jnp.clip(
        jnp.sum(degi[...], axis=0, keepdims=True), 1.0, None)))
    h2[...] = jnp.dot(feat[...], W[...], preferred_element_type=jnp.float32)


_tc_norms_h2 = pl.pallas_call(
    _tc_norms_h2_body,
    out_shape=(jax.ShapeDtypeStruct((N, 1), jnp.float32),
               jax.ShapeDtypeStruct((N, 1), jnp.float32),
               jax.ShapeDtypeStruct((N, D), jnp.float32)),
)


def _tc_norms_body(dego, degi, noc, nic):
    noc[...] = _colT(lax.rsqrt(jnp.clip(
        jnp.sum(dego[...], axis=0, keepdims=True), 1.0, None)))
    nic[...] = _colT(lax.rsqrt(jnp.clip(
        jnp.sum(degi[...], axis=0, keepdims=True), 1.0, None)))


_tc_norms = pl.pallas_call(
    _tc_norms_body,
    out_shape=(jax.ShapeDtypeStruct((N, 1), jnp.float32),
               jax.ShapeDtypeStruct((N, 1), jnp.float32)),
)


def _tc_out_s2_body(aggp, nic, b, Wp, out, s2):
    o = jnp.maximum((aggp[0] + aggp[1]) * nic[...] + b[...][None, :], 0.0)
    out[...] = o
    s2[...] = jnp.dot(o, Wp[...], preferred_element_type=jnp.float32)


_tc_out_s2 = pl.pallas_call(
    _tc_out_s2_body,
    out_shape=(jax.ShapeDtypeStruct((N, D), jnp.float32),
               jax.ShapeDtypeStruct((N, 1), jnp.float32)),
)


def _tc_pool0_body(saggp, nic, bp, out0, W1, keep_o, h2o, r0o):
    score = _colT(jnp.sum(saggp[...], axis=0, keepdims=True)) * nic[...] + bp[0]
    m = _sortable(score)
    thr = _kth_thr(m, KA)
    keep = m >= thr
    keepf = keep.astype(jnp.float32)
    nf = out0[...] * (jnp.tanh(score) * keepf)
    keep_o[...] = keepf
    h2o[...] = jnp.dot(nf, W1[...], preferred_element_type=jnp.float32)
    rmean = jnp.sum(nf, axis=0, keepdims=True) * (1.0 / KA)
    rmax = jnp.max(jnp.where(keep, nf, -jnp.inf), axis=0, keepdims=True)
    r0o[...] = jnp.concatenate([rmean, rmax], axis=1)


_tc_pool0 = pl.pallas_call(
    _tc_pool0_body,
    out_shape=(jax.ShapeDtypeStruct((N, 1), jnp.float32),
               jax.ShapeDtypeStruct((N, D), jnp.float32),
               jax.ShapeDtypeStruct((1, 2 * D), jnp.float32)),
)


def _tc_final_body(saggp, nic, bp, out1, keep0, r0,
                   W1, b1, W2, b2, W3, b3, out):
    score = _colT(jnp.sum(saggp[...], axis=0, keepdims=True)) * nic[...] + bp[0]
    m = _sortable(score)
    m = jnp.where(keep0[...] > 0.0, m, jnp.uint32(0))
    thr = _kth_thr(m, KB)
    keep = m >= thr
    keepf = keep.astype(jnp.float32)
    nf = out1[...] * (jnp.tanh(score) * keepf)
    rmean = jnp.sum(nf, axis=0, keepdims=True) * (1.0 / KB)
    rmax = jnp.max(jnp.where(keep, nf, -jnp.inf), axis=0, keepdims=True)
    fr = r0[...] + jnp.concatenate([rmean, rmax], axis=1)
    h = jnp.maximum(jnp.dot(fr, W1[...]) + b1[...][None, :], 0.0)
    h = jnp.maximum(jnp.dot(h, W2[...]) + b2[...][None, :], 0.0)
    z = jnp.dot(h, W3[...]) + b3[...][None, :]
    zm = z - jnp.max(z, axis=-1, keepdims=True)
    out[...] = zm - jnp.log(jnp.sum(jnp.exp(zm), axis=-1, keepdims=True))


_tc_final = pl.pallas_call(
    _tc_final_body, out_shape=jax.ShapeDtypeStruct((1, 2), jnp.float32))


def kernel(feat, edge_index, eweight, conv0_W, conv0_b, pool0_W, pool0_b,
           conv1_W, conv1_b, pool1_W, pool1_b, lin1_W, lin1_b, lin2_W, lin2_b,
           lin3_W, lin3_b):
    src = edge_index[0]
    dst = edge_index[1]
    ones = jnp.ones((N,), jnp.float32)

    dego0, degi0 = _sc_degrees(src, dst, ones)
    noc0, nic0, h2_0 = _tc_norms_h2(dego0.reshape(NW, N), degi0.reshape(NW, N),
                                    feat, conv0_W)
    no0 = noc0.reshape(N)
    aggp0 = _sc_edge_pass(h2_0, src, dst, eweight, no0, ones)
    out0, s2_0 = _tc_out_s2(aggp0, nic0, conv0_b, pool0_W)
    saggp0 = _sc_score_pass(src, dst, eweight, s2_0.reshape(N), no0, ones)
    keep0, h2_1, r0 = _tc_pool0(saggp0.reshape(NW, N), nic0, pool0_b,
                                out0, conv1_W)
    keep0f = keep0.reshape(N)

    dego1, degi1 = _sc_degrees(src, dst, keep0f)
    noc1, nic1 = _tc_norms(dego1.reshape(NW, N), degi1.reshape(NW, N))
    no1 = noc1.reshape(N)
    aggp1 = _sc_edge_pass(h2_1, src, dst, eweight, no1, keep0f)
    out1, s2_1 = _tc_out_s2(aggp1, nic1, conv1_b, pool1_W)
    saggp1 = _sc_score_pass(src, dst, eweight, s2_1.reshape(N), no1, keep0f)
    return _tc_final(saggp1.reshape(NW, N), nic1, pool1_b, out1, keep0, r0,
                     lin1_W, lin1_b, lin2_W, lin2_b, lin3_W, lin3_b)


# speedup vs baseline: 21.8638x; 1.1682x over previous
import functools

import jax
import jax.numpy as jnp
from jax import lax
from jax.experimental import pallas as pl
from jax.experimental.pallas import tpu as pltpu
from jax.experimental.pallas import tpu_sc as plsc

N = 10000
E = 320000
D = 128
NC = 2
NS = 16
NW = NC * NS
EPW = E // NW
C = 80
NCHUNK = EPW // C
TR = 624
KA = 5000
KB = 2500

_MESH = plsc.VectorSubcoreMesh(core_axis_name="c", subcore_axis_name="s")


def _z16():
    return jnp.zeros((16,), jnp.float32)


@functools.partial(
    pl.kernel,
    out_type=(jax.ShapeDtypeStruct((NW * N,), jnp.float32),
              jax.ShapeDtypeStruct((NW * N,), jnp.float32)),
    mesh=_MESH,
    compiler_params=pltpu.CompilerParams(needs_layout_passes=False),
    scratch_types=[
        pltpu.VMEM((EPW,), jnp.int32),
        pltpu.VMEM((EPW,), jnp.int32),
        pltpu.VMEM((N,), jnp.float32),
        pltpu.VMEM((N,), jnp.float32),
        pltpu.VMEM((N,), jnp.float32),
    ],
)
def _sc_degrees(src_hbm, dst_hbm, keep_hbm, dego_hbm, degi_hbm,
                src_v, dst_v, keep_v, dego_v, degi_v):
    wid = lax.axis_index("s") * NC + lax.axis_index("c")
    base = pl.multiple_of(wid * EPW, 8)
    pltpu.sync_copy(src_hbm.at[pl.ds(base, EPW)], src_v)
    pltpu.sync_copy(dst_hbm.at[pl.ds(base, EPW)], dst_v)
    pltpu.sync_copy(keep_hbm, keep_v)

    def zbody(i, _):
        dego_v[pl.ds(i * 16, 16)] = _z16()
        degi_v[pl.ds(i * 16, 16)] = _z16()
        return 0

    lax.fori_loop(0, N // 16, zbody, 0)

    def ebody(g, _):
        sl = pl.ds(g * 16, 16)
        s = src_v[sl]
        d = dst_v[sl]
        m = plsc.load_gather(keep_v, [s]) * plsc.load_gather(keep_v, [d])
        plsc.addupdate_scatter(dego_v, [s], m)
        plsc.addupdate_scatter(degi_v, [d], m)
        return 0

    lax.fori_loop(0, EPW // 16, ebody, 0)
    obase = pl.multiple_of(wid * N, 8)
    pltpu.sync_copy(dego_v, dego_hbm.at[pl.ds(obase, N)])
    pltpu.sync_copy(degi_v, degi_hbm.at[pl.ds(obase, N)])


@functools.partial(
    pl.kernel,
    out_type=jax.ShapeDtypeStruct((NC, N, D), jnp.float32),
    mesh=_MESH,
    compiler_params=pltpu.CompilerParams(needs_layout_passes=False),
    scratch_types=[
        pltpu.VMEM((N,), jnp.float32),
        pltpu.VMEM((N,), jnp.float32),
        pltpu.VMEM((C,), jnp.int32),
        pltpu.VMEM((C,), jnp.int32),
        pltpu.VMEM((C,), jnp.int32),
        pltpu.VMEM((C,), jnp.float32),
        pltpu.VMEM((C,), jnp.float32),
        pltpu.VMEM((C, D), jnp.float32),
        pltpu.VMEM((C, D), jnp.float32),
        pltpu.VMEM_SHARED((N, D), jnp.float32),
        pltpu.SemaphoreType.DMA,
        pltpu.SemaphoreType.DMA,
    ],
)
def _sc_edge_pass(h2_hbm, src_hbm, dst_hbm, ew_hbm, norm_hbm, keep_hbm, out_hbm,
                  norm_v, keep_v, srca_v, srcb_v, dstc_v, ewc_v, coef_v,
                  rowsa_v, rowsb_v, agg_sh, sema, semb):
    cid = lax.axis_index("c")
    sid = lax.axis_index("s")
    wid = sid * NC + cid
    base = pl.multiple_of(wid * EPW, 8)
    pltpu.sync_copy(norm_hbm, norm_v)
    pltpu.sync_copy(keep_hbm, keep_v)

    def z0(i, _):
        rowsa_v[i // 8, pl.ds((i % 8) * 16, 16)] = _z16()
        return 0

    lax.fori_loop(0, 16 * 8, z0, 0)
    zslab = rowsa_v.at[pl.ds(0, 16)]

    def z1(i, _):
        pltpu.sync_copy(zslab, agg_sh.at[pl.ds(sid * TR + i * 16, 16)])
        return 0

    lax.fori_loop(0, TR // 16, z1, 0)

    @pl.when(sid == NS - 1)
    def _():
        pltpu.sync_copy(zslab, agg_sh.at[pl.ds(NS * TR, N - NS * TR)])

    plsc.subcore_barrier()

    def fire(i, srcv, rowsv, sem):
        off = pl.multiple_of(base + i * C, 8)
        pltpu.sync_copy(src_hbm.at[pl.ds(off, C)], srcv)
        return pltpu.async_copy(h2_hbm.at[srcv], rowsv, sem)

    def process(i, srcv, rowsv):
        off = pl.multiple_of(base + i * C, 8)
        pltpu.sync_copy(dst_hbm.at[pl.ds(off, C)], dstc_v)
        pltpu.sync_copy(ew_hbm.at[pl.ds(off, C)], ewc_v)

        def cbody(g, _):
            sl = pl.ds(g * 16, 16)
            s = srcv[sl]
            d = dstc_v[sl]
            no = plsc.load_gather(norm_v, [s])
            ks = plsc.load_gather(keep_v, [s])
            kd = plsc.load_gather(keep_v, [d])
            coef_v[sl] = no * ewc_v[sl] * ks * kd
            return 0

        lax.fori_loop(0, C // 16, cbody, 0)

        def sbody(e, _):
            cvec = plsc.load_gather(coef_v, [jnp.full((16,), e, jnp.int32)])
            for j in range(D // 16):
                sl = pl.ds(j * 16, 16)
                rowsv[e, sl] = rowsv[e, sl] * cvec
            return 0

        lax.fori_loop(0, C, sbody, 0)
        pltpu.sync_copy(rowsv, agg_sh.at[dstc_v], add=True)

    def pair(j, _):
        i0 = j * 2
        cpa = fire(i0, srca_v, rowsa_v, sema)
        cpb = fire(i0 + 1, srcb_v, rowsb_v, semb)
        cpa.wait()
        process(i0, srca_v, rowsa_v)
        cpb.wait()
        process(i0 + 1, srcb_v, rowsb_v)
        return 0

    lax.fori_loop(0, NCHUNK // 2, pair, 0)
    fire(NCHUNK - 1, srca_v, rowsa_v, sema).wait()
    process(NCHUNK - 1, srca_v, rowsa_v)
    plsc.subcore_barrier()
    r0 = sid * TR
    pltpu.sync_copy(agg_sh.at[pl.ds(r0, TR)], out_hbm.at[cid, pl.ds(r0, TR)])

    @pl.when(sid == NS - 1)
    def _():
        pltpu.sync_copy(agg_sh.at[pl.ds(NS * TR, N - NS * TR)],
                        out_hbm.at[cid, pl.ds(NS * TR, N - NS * TR)])


@functools.partial(
    pl.kernel,
    out_type=jax.ShapeDtypeStruct((NW * N,), jnp.float32),
    mesh=_MESH,
    compiler_params=pltpu.CompilerParams(needs_layout_passes=False),
    scratch_types=[
        pltpu.VMEM((EPW,), jnp.int32),
        pltpu.VMEM((EPW,), jnp.int32),
        pltpu.VMEM((EPW,), jnp.float32),
        pltpu.VMEM((N,), jnp.float32),
        pltpu.VMEM((N,), jnp.float32),
        pltpu.VMEM((N,), jnp.float32),
        pltpu.VMEM((N,), jnp.float32),
    ],
)
def _sc_score_pass(src_hbm, dst_hbm, ew_hbm, s2_hbm, norm_hbm, keep_hbm, out_hbm,
                   src_v, dst_v, ew_v, s2_v, norm_v, keep_v, sagg_v):
    wid = lax.axis_index("s") * NC + lax.axis_index("c")
    base = pl.multiple_of(wid * EPW, 8)
    pltpu.sync_copy(src_hbm.at[pl.ds(base, EPW)], src_v)
    pltpu.sync_copy(dst_hbm.at[pl.ds(base, EPW)], dst_v)
    pltpu.sync_copy(ew_hbm.at[pl.ds(base, EPW)], ew_v)
    pltpu.sync_copy(s2_hbm, s2_v)
    pltpu.sync_copy(norm_hbm, norm_v)
    pltpu.sync_copy(keep_hbm, keep_v)

    def zbody(i, _):
        sagg_v[pl.ds(i * 16, 16)] = _z16()
        return 0

    lax.fori_loop(0, N // 16, zbody, 0)

    def ebody(g, _):
        sl = pl.ds(g * 16, 16)
        s = src_v[sl]
        d = dst_v[sl]
        val = (plsc.load_gather(s2_v, [s]) * plsc.load_gather(norm_v, [s])
               * ew_v[sl] * plsc.load_gather(keep_v, [s])
               * plsc.load_gather(keep_v, [d]))
        plsc.addupdate_scatter(sagg_v, [d], val)
        return 0

    lax.fori_loop(0, EPW // 16, ebody, 0)
    obase = pl.multiple_of(wid * N, 8)
    pltpu.sync_copy(sagg_v, out_hbm.at[pl.ds(obase, N)])


def _sortable(x):
    b = lax.bitcast_convert_type(x, jnp.uint32)
    return jnp.where((b >> jnp.uint32(31)) != jnp.uint32(0),
                     ~b, b | jnp.uint32(0x80000000))


def _kth_thr(m, k):
    thr = jnp.uint32(0)
    for i in range(31, -1, -1):
        cand = thr | jnp.uint32(1 << i)
        cnt = jnp.sum((m >= cand).astype(jnp.int32))
        thr = jnp.where(cnt >= k, cand, thr)
    return thr


def _colT(x):
    return lax.transpose(x, (1, 0))


def _tc_norms_h2_body(dego, degi, feat, W, noc, nic, h2):
    noc[...] = _colT(lax.rsqrt(jnp.clip(
        jnp.sum(dego[...], axis=0, keepdims=True), 1.0, None)))
    nic[...] = _colT(lax.rsqrt(jnp.clip(
        jnp.sum(degi[...], axis=0, keepdims=True), 1.0, None)))
    h2[...] = jnp.dot(feat[...], W[...], preferred_element_type=jnp.float32)


_tc_norms_h2 = pl.pallas_call(
    _tc_norms_h2_body,
    out_shape=(jax.ShapeDtypeStruct((N, 1), jnp.float32),
               jax.ShapeDtypeStruct((N, 1), jnp.float32),
               jax.ShapeDtypeStruct((N, D), jnp.float32)),
)


def _tc_norms_body(dego, degi, noc, nic):
    noc[...] = _colT(lax.rsqrt(jnp.clip(
        jnp.sum(dego[...], axis=0, keepdims=True), 1.0, None)))
    nic[...] = _colT(lax.rsqrt(jnp.clip(
        jnp.sum(degi[...], axis=0, keepdims=True), 1.0, None)))


_tc_norms = pl.pallas_call(
    _tc_norms_body,
    out_shape=(jax.ShapeDtypeStruct((N, 1), jnp.float32),
               jax.ShapeDtypeStruct((N, 1), jnp.float32)),
)


def _tc_out_s2_body(aggp, nic, b, Wp, out, s2):
    o = jnp.maximum((aggp[0] + aggp[1]) * nic[...] + b[...][None, :], 0.0)
    out[...] = o
    s2[...] = jnp.dot(o, Wp[...], preferred_element_type=jnp.float32)


_tc_out_s2 = pl.pallas_call(
    _tc_out_s2_body,
    out_shape=(jax.ShapeDtypeStruct((N, D), jnp.float32),
               jax.ShapeDtypeStruct((N, 1), jnp.float32)),
)


def _tc_pool0_body(saggp, nic, bp, out0, W1, keep_o, h2o, r0o):
    score = _colT(jnp.sum(saggp[...], axis=0, keepdims=True)) * nic[...] + bp[0]
    m = _sortable(score)
    thr = _kth_thr(m, KA)
    keep = m >= thr
    keepf = keep.astype(jnp.float32)
    nf = out0[...] * (jnp.tanh(score) * keepf)
    keep_o[...] = keepf
    h2o[...] = jnp.dot(nf, W1[...], preferred_element_type=jnp.float32)
    rmean = jnp.sum(nf, axis=0, keepdims=True) * (1.0 / KA)
    rmax = jnp.max(jnp.where(keep, nf, -jnp.inf), axis=0, keepdims=True)
    r0o[...] = jnp.concatenate([rmean, rmax], axis=1)


_tc_pool0 = pl.pallas_call(
    _tc_pool0_body,
    out_shape=(jax.ShapeDtypeStruct((N, 1), jnp.float32),
               jax.ShapeDtypeStruct((N, D), jnp.float32),
               jax.ShapeDtypeStruct((1, 2 * D), jnp.float32)),
)


def _tc_final_body(saggp, nic, bp, out1, keep0, r0,
                   W1, b1, W2, b2, W3, b3, out):
    score = _colT(jnp.sum(saggp[...], axis=0, keepdims=True)) * nic[...] + bp[0]
    m = _sortable(score)
    m = jnp.where(keep0[...] > 0.0, m, jnp.uint32(0))
    thr = _kth_thr(m, KB)
    keep = m >= thr
    keepf = keep.astype(jnp.float32)
    nf = out1[...] * (jnp.tanh(score) * keepf)
    rmean = jnp.sum(nf, axis=0, keepdims=True) * (1.0 / KB)
    rmax = jnp.max(jnp.where(keep, nf, -jnp.inf), axis=0, keepdims=True)
    fr = r0[...] + jnp.concatenate([rmean, rmax], axis=1)
    h = jnp.maximum(jnp.dot(fr, W1[...]) + b1[...][None, :], 0.0)
    h = jnp.maximum(jnp.dot(h, W2[...]) + b2[...][None, :], 0.0)
    z = jnp.dot(h, W3[...]) + b3[...][None, :]
    zm = z - jnp.max(z, axis=-1, keepdims=True)
    out[...] = zm - jnp.log(jnp.sum(jnp.exp(zm), axis=-1, keepdims=True))


_tc_final = pl.pallas_call(
    _tc_final_body, out_shape=jax.ShapeDtypeStruct((1, 2), jnp.float32))


def kernel(feat, edge_index, eweight, conv0_W, conv0_b, pool0_W, pool0_b,
           conv1_W, conv1_b, pool1_W, pool1_b, lin1_W, lin1_b, lin2_W, lin2_b,
           lin3_W, lin3_b):
    src = edge_index[0]
    dst = edge_index[1]
    ones = jnp.ones((N,), jnp.float32)

    dego0, degi0 = _sc_degrees(src, dst, ones)
    noc0, nic0, h2_0 = _tc_norms_h2(dego0.reshape(NW, N), degi0.reshape(NW, N),
                                    feat, conv0_W)
    no0 = noc0.reshape(N)
    aggp0 = _sc_edge_pass(h2_0, src, dst, eweight, no0, ones)
    out0, s2_0 = _tc_out_s2(aggp0, nic0, conv0_b, pool0_W)
    saggp0 = _sc_score_pass(src, dst, eweight, s2_0.reshape(N), no0, ones)
    keep0, h2_1, r0 = _tc_pool0(saggp0.reshape(NW, N), nic0, pool0_b,
                                out0, conv1_W)
    keep0f = keep0.reshape(N)

    dego1, degi1 = _sc_degrees(src, dst, keep0f)
    noc1, nic1 = _tc_norms(dego1.reshape(NW, N), degi1.reshape(NW, N))
    no1 = noc1.reshape(N)
    aggp1 = _sc_edge_pass(h2_1, src, dst, eweight, no1, keep0f)
    out1, s2_1 = _tc_out_s2(aggp1, nic1, conv1_b, pool1_W)
    saggp1 = _sc_score_pass(src, dst, eweight, s2_1.reshape(N), no1, keep0f)
    return _tc_final(saggp1.reshape(NW, N), nic1, pool1_b, out1, keep0, r0,
                     lin1_W, lin1_b, lin2_W, lin2_b, lin3_W, lin3_b)


# speedup vs baseline: 23.5753x; 1.0783x over previous
import functools

import jax
import jax.numpy as jnp
from jax import lax
from jax.experimental import pallas as pl
from jax.experimental.pallas import tpu as pltpu
from jax.experimental.pallas import tpu_sc as plsc

N = 10000
E = 320000
D = 128
NC = 2
NS = 16
NW = NC * NS
EPW = E // NW
C = 80
NCHUNK = EPW // C
TR = 624
KA = 5000
KB = 2500

_MESH = plsc.VectorSubcoreMesh(core_axis_name="c", subcore_axis_name="s")


def _z16():
    return jnp.zeros((16,), jnp.float32)


@functools.partial(
    pl.kernel,
    out_type=(jax.ShapeDtypeStruct((NW * N,), jnp.float32),
              jax.ShapeDtypeStruct((NW * N,), jnp.float32)),
    mesh=_MESH,
    compiler_params=pltpu.CompilerParams(needs_layout_passes=False),
    scratch_types=[
        pltpu.VMEM((EPW,), jnp.int32),
        pltpu.VMEM((EPW,), jnp.int32),
        pltpu.VMEM((N,), jnp.float32),
        pltpu.VMEM((N,), jnp.float32),
        pltpu.VMEM((N,), jnp.float32),
    ],
)
def _sc_degrees(src_hbm, dst_hbm, keep_hbm, dego_hbm, degi_hbm,
                src_v, dst_v, keep_v, dego_v, degi_v):
    wid = lax.axis_index("s") * NC + lax.axis_index("c")
    base = pl.multiple_of(wid * EPW, 8)
    pltpu.sync_copy(src_hbm.at[pl.ds(base, EPW)], src_v)
    pltpu.sync_copy(dst_hbm.at[pl.ds(base, EPW)], dst_v)
    pltpu.sync_copy(keep_hbm, keep_v)

    def zbody(i, _):
        dego_v[pl.ds(i * 16, 16)] = _z16()
        degi_v[pl.ds(i * 16, 16)] = _z16()
        return 0

    lax.fori_loop(0, N // 16, zbody, 0)

    def ebody(g, _):
        sl = pl.ds(g * 16, 16)
        s = src_v[sl]
        d = dst_v[sl]
        m = plsc.load_gather(keep_v, [s]) * plsc.load_gather(keep_v, [d])
        plsc.addupdate_scatter(dego_v, [s], m)
        plsc.addupdate_scatter(degi_v, [d], m)
        return 0

    lax.fori_loop(0, EPW // 16, ebody, 0)
    obase = pl.multiple_of(wid * N, 8)
    pltpu.sync_copy(dego_v, dego_hbm.at[pl.ds(obase, N)])
    pltpu.sync_copy(degi_v, degi_hbm.at[pl.ds(obase, N)])


@functools.partial(
    pl.kernel,
    out_type=jax.ShapeDtypeStruct((NC, N, D), jnp.float32),
    mesh=_MESH,
    compiler_params=pltpu.CompilerParams(needs_layout_passes=False),
    scratch_types=[
        pltpu.VMEM((N,), jnp.float32),
        pltpu.VMEM((N,), jnp.float32),
        pltpu.VMEM((C,), jnp.int32),
        pltpu.VMEM((C,), jnp.int32),
        pltpu.VMEM((C,), jnp.int32),
        pltpu.VMEM((C,), jnp.float32),
        pltpu.VMEM((C,), jnp.float32),
        pltpu.VMEM((C, D), jnp.float32),
        pltpu.VMEM((C, D), jnp.float32),
        pltpu.VMEM_SHARED((N, D), jnp.float32),
        pltpu.SemaphoreType.DMA,
        pltpu.SemaphoreType.DMA,
    ],
)
def _sc_edge_pass(h2_hbm, src_hbm, dst_hbm, ew_hbm, norm_hbm, keep_hbm, out_hbm,
                  norm_v, keep_v, srca_v, srcb_v, dstc_v, ewc_v, coef_v,
                  rowsa_v, rowsb_v, agg_sh, sema, semb):
    cid = lax.axis_index("c")
    sid = lax.axis_index("s")
    wid = sid * NC + cid
    base = pl.multiple_of(wid * EPW, 8)
    pltpu.sync_copy(norm_hbm, norm_v)
    pltpu.sync_copy(keep_hbm, keep_v)

    def z0(i, _):
        rowsa_v[i // 8, pl.ds((i % 8) * 16, 16)] = _z16()
        return 0

    lax.fori_loop(0, 16 * 8, z0, 0)
    zslab = rowsa_v.at[pl.ds(0, 16)]

    def z1(i, _):
        pltpu.sync_copy(zslab, agg_sh.at[pl.ds(sid * TR + i * 16, 16)])
        return 0

    lax.fori_loop(0, TR // 16, z1, 0)

    @pl.when(sid == NS - 1)
    def _():
        pltpu.sync_copy(zslab, agg_sh.at[pl.ds(NS * TR, N - NS * TR)])

    plsc.subcore_barrier()

    def fire(i, srcv, rowsv, sem):
        off = pl.multiple_of(base + i * C, 8)
        pltpu.sync_copy(src_hbm.at[pl.ds(off, C)], srcv)
        return pltpu.async_copy(h2_hbm.at[srcv], rowsv, sem)

    def process(i, srcv, rowsv):
        off = pl.multiple_of(base + i * C, 8)
        pltpu.sync_copy(dst_hbm.at[pl.ds(off, C)], dstc_v)
        pltpu.sync_copy(ew_hbm.at[pl.ds(off, C)], ewc_v)

        def cbody(g, _):
            sl = pl.ds(g * 16, 16)
            s = srcv[sl]
            d = dstc_v[sl]
            no = plsc.load_gather(norm_v, [s])
            ks = plsc.load_gather(keep_v, [s])
            kd = plsc.load_gather(keep_v, [d])
            coef_v[sl] = no * ewc_v[sl] * ks * kd
            return 0

        lax.fori_loop(0, C // 16, cbody, 0)

        def sbody(e, _):
            cvec = plsc.load_gather(coef_v, [jnp.full((16,), e, jnp.int32)])
            for j in range(D // 16):
                sl = pl.ds(j * 16, 16)
                rowsv[e, sl] = rowsv[e, sl] * cvec
            return 0

        lax.fori_loop(0, C, sbody, 0)
        pltpu.sync_copy(rowsv, agg_sh.at[dstc_v], add=True)

    nfull = NCHUNK // 2
    fire(0, srca_v, rowsa_v, sema)
    fire(1, srcb_v, rowsb_v, semb)

    def ring(j, _):
        pltpu.make_async_copy(h2_hbm.at[srca_v], rowsa_v, sema).wait()
        process(j * 2, srca_v, rowsa_v)

        @pl.when(j < nfull - 1)
        def _():
            fire(j * 2 + 2, srca_v, rowsa_v, sema)

        @pl.when(j == nfull - 1)
        def _():
            fire(NCHUNK - 1, srca_v, rowsa_v, sema)

        pltpu.make_async_copy(h2_hbm.at[srcb_v], rowsb_v, semb).wait()
        process(j * 2 + 1, srcb_v, rowsb_v)

        @pl.when(j < nfull - 1)
        def _():
            fire(j * 2 + 3, srcb_v, rowsb_v, semb)

        return 0

    lax.fori_loop(0, nfull, ring, 0)
    pltpu.make_async_copy(h2_hbm.at[srca_v], rowsa_v, sema).wait()
    process(NCHUNK - 1, srca_v, rowsa_v)
    plsc.subcore_barrier()
    r0 = sid * TR
    pltpu.sync_copy(agg_sh.at[pl.ds(r0, TR)], out_hbm.at[cid, pl.ds(r0, TR)])

    @pl.when(sid == NS - 1)
    def _():
        pltpu.sync_copy(agg_sh.at[pl.ds(NS * TR, N - NS * TR)],
                        out_hbm.at[cid, pl.ds(NS * TR, N - NS * TR)])


@functools.partial(
    pl.kernel,
    out_type=jax.ShapeDtypeStruct((NW * N,), jnp.float32),
    mesh=_MESH,
    compiler_params=pltpu.CompilerParams(needs_layout_passes=False),
    scratch_types=[
        pltpu.VMEM((EPW,), jnp.int32),
        pltpu.VMEM((EPW,), jnp.int32),
        pltpu.VMEM((EPW,), jnp.float32),
        pltpu.VMEM((N,), jnp.float32),
        pltpu.VMEM((N,), jnp.float32),
        pltpu.VMEM((N,), jnp.float32),
        pltpu.VMEM((N,), jnp.float32),
    ],
)
def _sc_score_pass(src_hbm, dst_hbm, ew_hbm, s2_hbm, norm_hbm, keep_hbm, out_hbm,
                   src_v, dst_v, ew_v, s2_v, norm_v, keep_v, sagg_v):
    wid = lax.axis_index("s") * NC + lax.axis_index("c")
    base = pl.multiple_of(wid * EPW, 8)
    pltpu.sync_copy(src_hbm.at[pl.ds(base, EPW)], src_v)
    pltpu.sync_copy(dst_hbm.at[pl.ds(base, EPW)], dst_v)
    pltpu.sync_copy(ew_hbm.at[pl.ds(base, EPW)], ew_v)
    pltpu.sync_copy(s2_hbm, s2_v)
    pltpu.sync_copy(norm_hbm, norm_v)
    pltpu.sync_copy(keep_hbm, keep_v)

    def zbody(i, _):
        sagg_v[pl.ds(i * 16, 16)] = _z16()
        return 0

    lax.fori_loop(0, N // 16, zbody, 0)

    def ebody(g, _):
        sl = pl.ds(g * 16, 16)
        s = src_v[sl]
        d = dst_v[sl]
        val = (plsc.load_gather(s2_v, [s]) * plsc.load_gather(norm_v, [s])
               * ew_v[sl] * plsc.load_gather(keep_v, [s])
               * plsc.load_gather(keep_v, [d]))
        plsc.addupdate_scatter(sagg_v, [d], val)
        return 0

    lax.fori_loop(0, EPW // 16, ebody, 0)
    obase = pl.multiple_of(wid * N, 8)
    pltpu.sync_copy(sagg_v, out_hbm.at[pl.ds(obase, N)])


def _sortable(x):
    b = lax.bitcast_convert_type(x, jnp.uint32)
    return jnp.where((b >> jnp.uint32(31)) != jnp.uint32(0),
                     ~b, b | jnp.uint32(0x80000000))


def _kth_thr(m, k):
    thr = jnp.uint32(0)
    for i in range(31, -1, -1):
        cand = thr | jnp.uint32(1 << i)
        cnt = jnp.sum((m >= cand).astype(jnp.int32))
        thr = jnp.where(cnt >= k, cand, thr)
    return thr


def _colT(x):
    return lax.transpose(x, (1, 0))


def _tc_norms_h2_body(dego, degi, feat, W, noc, nic, h2):
    noc[...] = _colT(lax.rsqrt(jnp.clip(
        jnp.sum(dego[...], axis=0, keepdims=True), 1.0, None)))
    nic[...] = _colT(lax.rsqrt(jnp.clip(
        jnp.sum(degi[...], axis=0, keepdims=True), 1.0, None)))
    h2[...] = jnp.dot(feat[...], W[...], preferred_element_type=jnp.float32)


_tc_norms_h2 = pl.pallas_call(
    _tc_norms_h2_body,
    out_shape=(jax.ShapeDtypeStruct((N, 1), jnp.float32),
               jax.ShapeDtypeStruct((N, 1), jnp.float32),
               jax.ShapeDtypeStruct((N, D), jnp.float32)),
)


def _tc_norms_body(dego, degi, noc, nic):
    noc[...] = _colT(lax.rsqrt(jnp.clip(
        jnp.sum(dego[...], axis=0, keepdims=True), 1.0, None)))
    nic[...] = _colT(lax.rsqrt(jnp.clip(
        jnp.sum(degi[...], axis=0, keepdims=True), 1.0, None)))


_tc_norms = pl.pallas_call(
    _tc_norms_body,
    out_shape=(jax.ShapeDtypeStruct((N, 1), jnp.float32),
               jax.ShapeDtypeStruct((N, 1), jnp.float32)),
)


def _tc_out_s2_body(aggp, nic, b, Wp, out, s2):
    o = jnp.maximum((aggp[0] + aggp[1]) * nic[...] + b[...][None, :], 0.0)
    out[...] = o
    s2[...] = jnp.dot(o, Wp[...], preferred_element_type=jnp.float32)


_tc_out_s2 = pl.pallas_call(
    _tc_out_s2_body,
    out_shape=(jax.ShapeDtypeStruct((N, D), jnp.float32),
               jax.ShapeDtypeStruct((N, 1), jnp.float32)),
)


def _tc_pool0_body(saggp, nic, bp, out0, W1, keep_o, h2o, r0o):
    score = _colT(jnp.sum(saggp[...], axis=0, keepdims=True)) * nic[...] + bp[0]
    m = _sortable(score)
    thr = _kth_thr(m, KA)
    keep = m >= thr
    keepf = keep.astype(jnp.float32)
    nf = out0[...] * (jnp.tanh(score) * keepf)
    keep_o[...] = keepf
    h2o[...] = jnp.dot(nf, W1[...], preferred_element_type=jnp.float32)
    rmean = jnp.sum(nf, axis=0, keepdims=True) * (1.0 / KA)
    rmax = jnp.max(jnp.where(keep, nf, -jnp.inf), axis=0, keepdims=True)
    r0o[...] = jnp.concatenate([rmean, rmax], axis=1)


_tc_pool0 = pl.pallas_call(
    _tc_pool0_body,
    out_shape=(jax.ShapeDtypeStruct((N, 1), jnp.float32),
               jax.ShapeDtypeStruct((N, D), jnp.float32),
               jax.ShapeDtypeStruct((1, 2 * D), jnp.float32)),
)


def _tc_final_body(saggp, nic, bp, out1, keep0, r0,
                   W1, b1, W2, b2, W3, b3, out):
    score = _colT(jnp.sum(saggp[...], axis=0, keepdims=True)) * nic[...] + bp[0]
    m = _sortable(score)
    m = jnp.where(keep0[...] > 0.0, m, jnp.uint32(0))
    thr = _kth_thr(m, KB)
    keep = m >= thr
    keepf = keep.astype(jnp.float32)
    nf = out1[...] * (jnp.tanh(score) * keepf)
    rmean = jnp.sum(nf, axis=0, keepdims=True) * (1.0 / KB)
    rmax = jnp.max(jnp.where(keep, nf, -jnp.inf), axis=0, keepdims=True)
    fr = r0[...] + jnp.concatenate([rmean, rmax], axis=1)
    h = jnp.maximum(jnp.dot(fr, W1[...]) + b1[...][None, :], 0.0)
    h = jnp.maximum(jnp.dot(h, W2[...]) + b2[...][None, :], 0.0)
    z = jnp.dot(h, W3[...]) + b3[...][None, :]
    zm = z - jnp.max(z, axis=-1, keepdims=True)
    out[...] = zm - jnp.log(jnp.sum(jnp.exp(zm), axis=-1, keepdims=True))


_tc_final = pl.pallas_call(
    _tc_final_body, out_shape=jax.ShapeDtypeStruct((1, 2), jnp.float32))


def kernel(feat, edge_index, eweight, conv0_W, conv0_b, pool0_W, pool0_b,
           conv1_W, conv1_b, pool1_W, pool1_b, lin1_W, lin1_b, lin2_W, lin2_b,
           lin3_W, lin3_b):
    src = edge_index[0]
    dst = edge_index[1]
    ones = jnp.ones((N,), jnp.float32)

    dego0, degi0 = _sc_degrees(src, dst, ones)
    noc0, nic0, h2_0 = _tc_norms_h2(dego0.reshape(NW, N), degi0.reshape(NW, N),
                                    feat, conv0_W)
    no0 = noc0.reshape(N)
    aggp0 = _sc_edge_pass(h2_0, src, dst, eweight, no0, ones)
    out0, s2_0 = _tc_out_s2(aggp0, nic0, conv0_b, pool0_W)
    saggp0 = _sc_score_pass(src, dst, eweight, s2_0.reshape(N), no0, ones)
    keep0, h2_1, r0 = _tc_pool0(saggp0.reshape(NW, N), nic0, pool0_b,
                                out0, conv1_W)
    keep0f = keep0.reshape(N)

    dego1, degi1 = _sc_degrees(src, dst, keep0f)
    noc1, nic1 = _tc_norms(dego1.reshape(NW, N), degi1.reshape(NW, N))
    no1 = noc1.reshape(N)
    aggp1 = _sc_edge_pass(h2_1, src, dst, eweight, no1, keep0f)
    out1, s2_1 = _tc_out_s2(aggp1, nic1, conv1_b, pool1_W)
    saggp1 = _sc_score_pass(src, dst, eweight, s2_1.reshape(N), no1, keep0f)
    return _tc_final(saggp1.reshape(NW, N), nic1, pool1_b, out1, keep0, r0,
                     lin1_W, lin1_b, lin2_W, lin2_b, lin3_W, lin3_b)
